# Initial kernel scaffold; baseline (speedup 1.0000x reference)
import jax
import jax.numpy as jnp
from jax.experimental import pallas as pl


def kernel(sources):
    raise NotImplementedError("write your pallas kernel here")


# speedup vs baseline: 2.2104x; 2.2104x over previous
import jax
import jax.numpy as jnp
from jax.experimental import pallas as pl
from jax.experimental.pallas import tpu as pltpu


def _copy_kernel(perm_ref, in_ref, out_ref):
    out_ref[...] = in_ref[...]


def kernel(sources):
    s2, bs, c, t = sources.shape
    perm_key = jax.random.key(42)
    perm = jnp.argsort(jax.random.uniform(perm_key, (bs,))).astype(jnp.int32)

    grid = (s2, bs)

    def in_index(s, b, perm_ref):
        row = jnp.where(s == 0, perm_ref[b], b)
        return (s, row, 0, 0)

    def out_index(s, b, perm_ref):
        return (s, b, 0, 0)

    return pl.pallas_call(
        _copy_kernel,
        grid_spec=pltpu.PrefetchScalarGridSpec(
            num_scalar_prefetch=1,
            grid=grid,
            in_specs=[pl.BlockSpec((1, 1, c, t), in_index)],
            out_specs=pl.BlockSpec((1, 1, c, t), out_index),
        ),
        out_shape=jax.ShapeDtypeStruct(sources.shape, sources.dtype),
    )(perm, sources)
